# baseline (speedup 1.0000x reference)
import jax
import jax.numpy as jnp
from jax.experimental import pallas as pl
from jax.experimental.pallas import tpu as pltpu


def _pos_embed_body(cw_ref, rw_ref, o_ref):
    cwT = cw_ref[...].T
    rwT = rw_ref[...].T
    d, w = cwT.shape
    h = rwT.shape[1]
    o_ref[0, 0:d] = jnp.broadcast_to(cwT[:, None, :], (d, h, w))
    o_ref[0, d:] = jnp.broadcast_to(rwT[:, :, None], (d, h, w))


def kernel(mask, row_weight, col_weight):
    bs, h, w = mask.shape
    d = row_weight.shape[1]
    out_shape = jax.ShapeDtypeStruct((bs, 2 * d, h, w), jnp.float32)
    return pl.pallas_call(
        _pos_embed_body,
        grid=(bs,),
        in_specs=[
            pl.BlockSpec((w, d), lambda b: (0, 0)),
            pl.BlockSpec((h, d), lambda b: (0, 0)),
        ],
        out_specs=pl.BlockSpec((1, 2 * d, h, w), lambda b: (b, 0, 0, 0)),
        out_shape=out_shape,
        compiler_params=pltpu.CompilerParams(
            dimension_semantics=(pltpu.PARALLEL,),
        ),
    )(col_weight, row_weight)


# speedup vs baseline: 1.0091x; 1.0091x over previous
import jax
import jax.numpy as jnp
from jax.experimental import pallas as pl
from jax.experimental.pallas import tpu as pltpu

_CCHUNKS = 8


def _pos_embed_body(cw_ref, rw_ref, o_ref, slab, sems):
    cwT = cw_ref[...].T
    rwT = rw_ref[...].T
    d, w = cwT.shape
    h = rwT.shape[1]
    slab[0:d] = jnp.broadcast_to(cwT[:, None, :], (d, h, w))
    slab[d:] = jnp.broadcast_to(rwT[:, :, None], (d, h, w))
    bs = o_ref.shape[0]
    csz = (2 * d) // _CCHUNKS
    for b in range(bs):
        for c in range(_CCHUNKS):
            pltpu.make_async_copy(
                slab.at[pl.ds(c * csz, csz)],
                o_ref.at[b, pl.ds(c * csz, csz)],
                sems.at[b, c],
            ).start()
    for b in range(bs):
        for c in range(_CCHUNKS):
            pltpu.make_async_copy(
                slab.at[pl.ds(c * csz, csz)],
                o_ref.at[b, pl.ds(c * csz, csz)],
                sems.at[b, c],
            ).wait()


def kernel(mask, row_weight, col_weight):
    bs, h, w = mask.shape
    d = row_weight.shape[1]
    out_shape = jax.ShapeDtypeStruct((bs, 2 * d, h, w), jnp.float32)
    return pl.pallas_call(
        _pos_embed_body,
        in_specs=[
            pl.BlockSpec(memory_space=pltpu.VMEM),
            pl.BlockSpec(memory_space=pltpu.VMEM),
        ],
        out_specs=pl.BlockSpec(memory_space=pl.ANY),
        out_shape=out_shape,
        scratch_shapes=[
            pltpu.VMEM((2 * d, h, w), jnp.float32),
            pltpu.SemaphoreType.DMA((bs, _CCHUNKS)),
        ],
    )(col_weight, row_weight)
